# baseline (speedup 1.0000x reference)
import functools

import jax
import jax.numpy as jnp
from jax import lax
from jax.experimental import pallas as pl
from jax.experimental.pallas import tpu as pltpu
from jax.experimental.pallas import tpu_sc as plsc

BATCH = 4096
HIST = 50
EMBED_DIM = 64
TOTAL = BATCH * HIST

NUM_CORES = 2
NUM_SUBCORES = 16
NUM_WORKERS = NUM_CORES * NUM_SUBCORES
ROWS_PER_WORKER = TOTAL // NUM_WORKERS
CHUNK = 128
N_CHUNKS = ROWS_PER_WORKER // CHUNK
NBUF = 10


def _build():
    mesh = plsc.VectorSubcoreMesh(core_axis_name="c", subcore_axis_name="s")

    @functools.partial(
        pl.kernel,
        mesh=mesh,
        out_type=jax.ShapeDtypeStruct((TOTAL, EMBED_DIM), jnp.float32),
        scratch_types=[
            pltpu.VMEM((N_CHUNKS, CHUNK), jnp.int32),
            pltpu.VMEM((NBUF, CHUNK, EMBED_DIM), jnp.float32),
            pltpu.SemaphoreType.DMA((NBUF,)),
        ],
        compiler_params=pltpu.CompilerParams(use_tc_tiling_on_sc=False),
    )
    def gather_kernel(idx_hbm, table_hbm, out_hbm, idx_v, rows_v, sems):
        wid = lax.axis_index("s") * NUM_CORES + lax.axis_index("c")
        base = wid * ROWS_PER_WORKER

        pltpu.sync_copy(idx_hbm.at[wid], idx_v)

        for b in range(NBUF):
            pltpu.async_copy(table_hbm.at[idx_v.at[b]], rows_v.at[b],
                             sems.at[b])

        def group(g, _):
            for b in range(NBUF):
                c = g * NBUF + b
                pltpu.make_async_copy(table_hbm.at[idx_v.at[b]],
                                      rows_v.at[b], sems.at[b]).wait()
                pltpu.sync_copy(rows_v.at[b],
                                out_hbm.at[pl.ds(base + c * CHUNK, CHUNK)])
                nxt = c + NBUF

                @pl.when(nxt < N_CHUNKS)
                def _():
                    pltpu.async_copy(table_hbm.at[idx_v.at[nxt]],
                                     rows_v.at[b], sems.at[b])

            return ()

        lax.fori_loop(0, N_CHUNKS // NBUF, group, (), unroll=False)

    return gather_kernel


_gather = _build()


@jax.jit
def kernel(x, table):
    idx3d = x.reshape(NUM_WORKERS, N_CHUNKS, CHUNK)
    out = _gather(idx3d, table)
    return out.reshape(BATCH, HIST, EMBED_DIM)


# speedup vs baseline: 1.0065x; 1.0065x over previous
import functools

import jax
import jax.numpy as jnp
from jax import lax
from jax.experimental import pallas as pl
from jax.experimental.pallas import tpu as pltpu
from jax.experimental.pallas import tpu_sc as plsc

BATCH = 4096
HIST = 50
EMBED_DIM = 64

NUM_CORES = 2
NUM_SUBCORES = 16
NUM_WORKERS = NUM_CORES * NUM_SUBCORES
BATCH_PER_WORKER = BATCH // NUM_WORKERS
NBUF = 8


def _build():
    mesh = plsc.VectorSubcoreMesh(core_axis_name="c", subcore_axis_name="s")

    @functools.partial(
        pl.kernel,
        mesh=mesh,
        out_type=jax.ShapeDtypeStruct((BATCH, HIST, EMBED_DIM), jnp.float32),
        scratch_types=[
            pltpu.VMEM((BATCH_PER_WORKER, HIST), jnp.int32),
            pltpu.VMEM((NBUF, HIST, EMBED_DIM), jnp.float32),
            pltpu.SemaphoreType.DMA((NBUF,)),
        ],
        compiler_params=pltpu.CompilerParams(use_tc_tiling_on_sc=False),
    )
    def gather_kernel(idx_hbm, table_hbm, out_hbm, idx_v, rows_v, sems):
        wid = lax.axis_index("s") * NUM_CORES + lax.axis_index("c")
        base = wid * BATCH_PER_WORKER

        pltpu.sync_copy(idx_hbm.at[pl.ds(base, BATCH_PER_WORKER)], idx_v)

        for b in range(NBUF):
            pltpu.async_copy(table_hbm.at[idx_v.at[b]], rows_v.at[b],
                             sems.at[b])

        def group(g, _):
            for b in range(NBUF):
                c = g * NBUF + b
                pltpu.make_async_copy(table_hbm.at[idx_v.at[b]],
                                      rows_v.at[b], sems.at[b]).wait()
                pltpu.sync_copy(rows_v.at[b], out_hbm.at[base + c])
                nxt = c + NBUF

                @pl.when(nxt < BATCH_PER_WORKER)
                def _():
                    pltpu.async_copy(table_hbm.at[idx_v.at[nxt]],
                                     rows_v.at[b], sems.at[b])

            return ()

        lax.fori_loop(0, BATCH_PER_WORKER // NBUF, group, (), unroll=False)

    return gather_kernel


_gather = _build()


@jax.jit
def kernel(x, table):
    return _gather(x, table)


# speedup vs baseline: 1.5083x; 1.4985x over previous
import functools

import jax
import jax.numpy as jnp
from jax import lax
from jax.experimental import pallas as pl
from jax.experimental.pallas import tpu as pltpu
from jax.experimental.pallas import tpu_sc as plsc

BATCH = 4096
HIST = 50
EMBED_DIM = 64
HIST_PAD = 56

NUM_CORES = 2
NUM_SUBCORES = 16
NUM_WORKERS = NUM_CORES * NUM_SUBCORES
BATCH_PER_WORKER = BATCH // NUM_WORKERS
IDX_ROWS_PER_WORKER = BATCH_PER_WORKER * HIST // 128
NBUF = 8


def _build():
    mesh = plsc.VectorSubcoreMesh(core_axis_name="c", subcore_axis_name="s")

    @functools.partial(
        pl.kernel,
        mesh=mesh,
        out_type=jax.ShapeDtypeStruct((BATCH, HIST_PAD, 128), jnp.float32),
        scratch_types=[
            pltpu.VMEM((BATCH_PER_WORKER, HIST), jnp.int32),
            pltpu.VMEM((NBUF, HIST, EMBED_DIM), jnp.float32),
            pltpu.SemaphoreType.DMA((NBUF,)),
        ],
        compiler_params=pltpu.CompilerParams(use_tc_tiling_on_sc=False),
    )
    def gather_kernel(idx_hbm, table_hbm, out_hbm, idx_v, rows_v, sems):
        wid = lax.axis_index("s") * NUM_CORES + lax.axis_index("c")
        base = wid * BATCH_PER_WORKER

        pltpu.sync_copy(idx_hbm.at[pl.ds(base, BATCH_PER_WORKER)], idx_v)

        for b in range(NBUF):
            pltpu.async_copy(table_hbm.at[idx_v.at[b]], rows_v.at[b],
                             sems.at[b])

        def group(g, _):
            for b in range(NBUF):
                c = g * NBUF + b
                pltpu.make_async_copy(table_hbm.at[idx_v.at[b]],
                                      rows_v.at[b], sems.at[b]).wait()
                pltpu.sync_copy(
                    rows_v.at[b],
                    out_hbm.at[base + c, pl.ds(0, HIST), pl.ds(0, EMBED_DIM)])
                nxt = c + NBUF

                @pl.when(nxt < BATCH_PER_WORKER)
                def _():
                    pltpu.async_copy(table_hbm.at[idx_v.at[nxt]],
                                     rows_v.at[b], sems.at[b])

            return ()

        lax.fori_loop(0, BATCH_PER_WORKER // NBUF, group, (), unroll=False)

    return gather_kernel


_gather = _build()


@jax.jit
def kernel(x, table):
    out = _gather(x, table)
    return out[:, :HIST, :EMBED_DIM]
